# baseline (speedup 1.0000x reference)
import functools

import jax
import jax.numpy as jnp
from jax import lax
from jax.experimental import pallas as pl
from jax.experimental.pallas import tpu as pltpu
from jax.experimental.pallas import tpu_sc as plsc

D_MODEL = 4096
N_EXP = 64
TOP_K = 8
N_TOK = 8192
BLK_T = 512
CHUNK = 256

_NEG_INF = float("-inf")


def _tc_kernel(h_ref, w_ref, probs_ref, logits_ref):
    logits = jax.lax.dot_general(
        h_ref[...], w_ref[...], (((1,), (1,)), ((), ())),
        preferred_element_type=jnp.float32,
    )
    logits_ref[...] = logits
    for c in range(BLK_T // CHUNK):
        sl = pl.ds(c * CHUNK, CHUNK)
        lg = logits[c * CHUNK:(c + 1) * CHUNK, :].T
        m = jnp.max(lg, axis=0, keepdims=True)
        e = jnp.exp(lg - m)
        probs_ref[sl, :] = (e / jnp.sum(e, axis=0, keepdims=True)).T


def _tc_call(h, W):
    grid = (N_TOK // BLK_T,)
    return pl.pallas_call(
        _tc_kernel,
        grid=grid,
        in_specs=[
            pl.BlockSpec((BLK_T, D_MODEL), lambda i: (i, 0)),
            pl.BlockSpec((N_EXP, D_MODEL), lambda i: (0, 0)),
        ],
        out_specs=[
            pl.BlockSpec((BLK_T, N_EXP), lambda i: (i, 0)),
            pl.BlockSpec((BLK_T, N_EXP), lambda i: (i, 0)),
        ],
        out_shape=[
            jax.ShapeDtypeStruct((N_TOK, N_EXP), jnp.float32),
            jax.ShapeDtypeStruct((N_TOK, N_EXP), jnp.float32),
        ],
        compiler_params=pltpu.CompilerParams(
            dimension_semantics=("parallel",),
        ),
    )(h, W)


_INFO = plsc.get_sparse_core_info()
_NC, _NS, _L = _INFO.num_cores, _INFO.num_subcores, _INFO.num_lanes
_NW = _NC * _NS
_TPW = N_TOK // _NW


def _sc_topk_kernel(logits_hbm, mask_hbm, lg_v, mk_v):
    wid = lax.axis_index("s") * _NC + lax.axis_index("c")
    base = wid * _TPW
    pltpu.sync_copy(logits_hbm.at[pl.ds(base, _TPW)], lg_v)
    lane = lax.iota(jnp.int32, _L)

    def body(t, carry):
        def dsort(v):
            return plsc.sort_key_val(v, v, descending=True)[0]

        vs = [lg_v[t, pl.ds(16 * j, 16)] for j in range(4)]
        ss = [dsort(v) for v in vs]
        m01 = dsort(jnp.maximum(ss[0], lax.rev(ss[1], (0,))))
        m23 = dsort(jnp.maximum(ss[2], lax.rev(ss[3], (0,))))
        t16 = dsort(jnp.maximum(m01, lax.rev(m23, (0,))))
        thr = jnp.max(jnp.where(lane == (TOP_K - 1), t16, _NEG_INF))
        for j in range(4):
            mk_v[t, pl.ds(16 * j, 16)] = jnp.where(vs[j] >= thr, 1.0, 0.0)
        return carry

    lax.fori_loop(0, _TPW, body, 0)
    pltpu.sync_copy(mk_v, mask_hbm.at[pl.ds(base, _TPW)])


@functools.partial(jax.jit, static_argnames=())
def kernel(h, W):
    probs, logits = _tc_call(h, W)
    mesh = plsc.VectorSubcoreMesh(core_axis_name="c", subcore_axis_name="s")
    maskf = pl.kernel(
        _sc_topk_kernel,
        mesh=mesh,
        out_type=jax.ShapeDtypeStruct((N_TOK, N_EXP), jnp.float32),
        scratch_types=[
            pltpu.VMEM((_TPW, N_EXP), jnp.float32),
            pltpu.VMEM((_TPW, N_EXP), jnp.float32),
        ],
        compiler_params=pltpu.CompilerParams(needs_layout_passes=False),
    )(logits)
    return (maskf.astype(jnp.bool_), probs, probs, logits)
